# Initial kernel scaffold; baseline (speedup 1.0000x reference)
import jax
import jax.numpy as jnp
from jax.experimental import pallas as pl


def kernel(x, edge_index, lin_w, lin_b, tc1_w1, tc1_b1, tc1_w2, tc1_b2, tc1_w3, tc1_b3, cheb_w, cheb_b, tc2_w1, tc2_b1, tc2_w2, tc2_b2, tc2_w3, tc2_b3):
    raise NotImplementedError("write your pallas kernel here")


# speedup vs baseline: 158.0967x; 158.0967x over previous
import functools

import jax
import jax.numpy as jnp
from jax import lax
from jax.experimental import pallas as pl
from jax.experimental.pallas import tpu as pltpu
from jax.experimental.pallas import tpu_sc as plsc

N = 10000
F_IN = 128
C = 32
T_OUT = 8

N_PAD = 10240
PADV = N
E = 320000
E2 = 2 * E
NC, NS, L = 2, 16, 16
NW = NC * NS
CHUNK = 128
EDGES_PER_W = 20480
E_PAD = NW * EDGES_PER_W
NCHUNKS = EDGES_PER_W // CHUNK
ROWS_PER_TILE = N_PAD // NS

def _sc_mesh():
  return plsc.VectorSubcoreMesh(
      core_axis_name="c", subcore_axis_name="s", num_cores=NC, num_subcores=NS)


def _hist_body(idx_hbm, out_hbm, idxv, ones_v, zbuf, acc_sh):
  c = lax.axis_index("c")
  s = lax.axis_index("s")
  wid = s * NC + c

  def fill_ones(i, _):
    ones_v[i, :] = jnp.full((L,), 1.0, jnp.float32)
    return 0
  lax.fori_loop(0, CHUNK, fill_ones, 0)

  def fill_z(i, _):
    zbuf[i, :] = jnp.zeros((L,), jnp.float32)
    return 0
  lax.fori_loop(0, ROWS_PER_TILE, fill_z, 0)
  pltpu.sync_copy(zbuf, acc_sh.at[pl.ds(s * ROWS_PER_TILE, ROWS_PER_TILE)])
  plsc.subcore_barrier()

  base = wid * EDGES_PER_W

  def body(k, _):
    pltpu.sync_copy(idx_hbm.at[pl.ds(base + k * CHUNK, CHUNK)], idxv)
    pltpu.sync_copy(ones_v, acc_sh.at[idxv], add=True)
    return 0
  lax.fori_loop(0, NCHUNKS, body, 0)
  plsc.subcore_barrier()

  pltpu.sync_copy(
      acc_sh.at[pl.ds(s * ROWS_PER_TILE, ROWS_PER_TILE)],
      out_hbm.at[pl.ds(c * N_PAD + s * ROWS_PER_TILE, ROWS_PER_TILE)])


@functools.cache
def _hist():
  return pl.kernel(
      _hist_body,
      out_type=jax.ShapeDtypeStruct((NC * N_PAD, L), jnp.float32),
      mesh=_sc_mesh(),
      scratch_types=[
          pltpu.VMEM((CHUNK,), jnp.int32),
          pltpu.VMEM((CHUNK, L), jnp.float32),
          pltpu.VMEM((ROWS_PER_TILE, L), jnp.float32),
          pltpu.VMEM_SHARED((N_PAD, L), jnp.float32),
      ],
      compiler_params=pltpu.CompilerParams(use_tc_tiling_on_sc=False),
  )


def _hop_body(row_hbm, col_hbm, tab_hbm, out_hbm,
              ridx, cidx, rows_v, zbuf, acc_sh, sem):
  c = lax.axis_index("c")
  s = lax.axis_index("s")
  wid = s * NC + c

  def fill(i, _):
    zbuf[i, pl.ds(0, L)] = jnp.zeros((L,), jnp.float32)
    zbuf[i, pl.ds(L, L)] = jnp.zeros((L,), jnp.float32)
    return 0
  lax.fori_loop(0, ROWS_PER_TILE, fill, 0)
  pltpu.sync_copy(zbuf, acc_sh.at[pl.ds(s * ROWS_PER_TILE, ROWS_PER_TILE)])
  plsc.subcore_barrier()

  base = wid * EDGES_PER_W

  def body(k, _):
    pltpu.sync_copy(row_hbm.at[pl.ds(base + k * CHUNK, CHUNK)], ridx)
    pltpu.sync_copy(col_hbm.at[pl.ds(base + k * CHUNK, CHUNK)], cidx)
    pltpu.async_copy(tab_hbm.at[ridx], rows_v, sem).wait()
    pltpu.sync_copy(rows_v, acc_sh.at[cidx], add=True)
    return 0
  lax.fori_loop(0, NCHUNKS, body, 0)
  plsc.subcore_barrier()

  pltpu.sync_copy(
      acc_sh.at[pl.ds(s * ROWS_PER_TILE, ROWS_PER_TILE)],
      out_hbm.at[pl.ds(c * N_PAD + s * ROWS_PER_TILE, ROWS_PER_TILE)])


@functools.cache
def _hop():
  return pl.kernel(
      _hop_body,
      out_type=jax.ShapeDtypeStruct((NC * N_PAD, C), jnp.float32),
      mesh=_sc_mesh(),
      scratch_types=[
          pltpu.VMEM((CHUNK,), jnp.int32),
          pltpu.VMEM((CHUNK,), jnp.int32),
          pltpu.VMEM((CHUNK, C), jnp.float32),
          pltpu.VMEM((ROWS_PER_TILE, C), jnp.float32),
          pltpu.VMEM_SHARED((N_PAD, C), jnp.float32),
          pltpu.SemaphoreType.DMA,
      ],
      compiler_params=pltpu.CompilerParams(use_tc_tiling_on_sc=False),
  )


def _sigmoid(x):
  return 1.0 / (1.0 + jnp.exp(-x))


def _tc1_body(x_ref, hist_ref, lw_ref, lb_ref, a1_ref, b1_ref, a2_ref,
              b2_ref, a3_ref, b3_ref, g_ref, v1_ref, disb_ref):
  x = x_ref[...]
  h = jnp.dot(x, lw_ref[...], preferred_element_type=jnp.float32) + lb_ref[...]
  p = jnp.dot(h, a1_ref[...], preferred_element_type=jnp.float32) + b1_ref[...]
  q = jnp.dot(h, a2_ref[...], preferred_element_type=jnp.float32) + b2_ref[...]
  r = jnp.dot(h, a3_ref[...], preferred_element_type=jnp.float32) + b3_ref[...]
  g = jnp.maximum(p * _sigmoid(q) + r, 0.0)

  hist = hist_ref[...]
  deg16 = hist[:N_PAD, :] + hist[N_PAD:, :]
  rowid16 = lax.broadcasted_iota(jnp.int32, (N_PAD, L), 0)
  dis16 = jnp.where((deg16 > 0.0) & (rowid16 < N), lax.rsqrt(deg16), 0.0)
  disb = jnp.concatenate([dis16, dis16], axis=1)

  rowid = lax.broadcasted_iota(jnp.int32, (N_PAD, C), 0)
  g = jnp.where(rowid < N, g, 0.0)
  g_ref[...] = g
  v1_ref[...] = disb * g
  disb_ref[...] = disb


def _tc2_body(p_ref, disb_ref, tx1_ref, v2_ref):
  p = p_ref[...]
  disb = disb_ref[...]
  u1 = p[:N_PAD, :] + p[N_PAD:, :]
  tx1 = -disb * u1
  tx1_ref[...] = tx1
  v2_ref[...] = disb * tx1


def _tc3_body(q_ref, disb_ref, g_ref, tx1_ref, w0_ref, w1_ref, w2_ref,
              cb_ref, d1_ref, e1_ref, d2_ref, e2_ref, d3_ref, e3_ref, o_ref):
  q = q_ref[...]
  disb = disb_ref[...]
  g = g_ref[...]
  tx1 = tx1_ref[...]
  u2 = q[:N_PAD, :] + q[N_PAD:, :]
  tx2 = -2.0 * disb * u2 - g
  cheb = (jnp.dot(g, w0_ref[...], preferred_element_type=jnp.float32)
          + jnp.dot(tx1, w1_ref[...], preferred_element_type=jnp.float32)
          + jnp.dot(tx2, w2_ref[...], preferred_element_type=jnp.float32)
          + cb_ref[...])
  cheb = jnp.maximum(cheb, 0.0)
  p2 = jnp.dot(cheb, d1_ref[...], preferred_element_type=jnp.float32) + e1_ref[...]
  q2 = jnp.dot(cheb, d2_ref[...], preferred_element_type=jnp.float32) + e2_ref[...]
  r2 = jnp.dot(cheb, d3_ref[...], preferred_element_type=jnp.float32) + e3_ref[...]
  o_ref[...] = jnp.maximum(p2 * _sigmoid(q2) + r2, 0.0)


_f32 = lambda *shape: jax.ShapeDtypeStruct(shape, jnp.float32)

_tc1 = pl.pallas_call(
    _tc1_body, out_shape=[_f32(N_PAD, C), _f32(N_PAD, C), _f32(N_PAD, C)])
_tc2 = pl.pallas_call(_tc2_body, out_shape=[_f32(N_PAD, C), _f32(N_PAD, C)])
_tc3 = pl.pallas_call(_tc3_body, out_shape=_f32(N_PAD, C))


def _eff(w):
  return jnp.transpose(jnp.sum(w[:, :, 0, :], axis=-1), (1, 0))


def kernel(x, edge_index, lin_w, lin_b, tc1_w1, tc1_b1, tc1_w2, tc1_b2,
           tc1_w3, tc1_b3, cheb_w, cheb_b, tc2_w1, tc2_b1, tc2_w2, tc2_b2,
           tc2_w3, tc2_b3):
  xp = jnp.pad(x, ((0, N_PAD - N), (0, 0)))
  ei0 = edge_index[0]
  ei1 = edge_index[1]
  pad = jnp.full((E_PAD - E2,), PADV, jnp.int32)
  rowp = jnp.concatenate([ei0, ei1, pad])
  colp = jnp.concatenate([ei1, ei0, pad])

  hist = _hist()(rowp)
  g, v1, disb = _tc1(
      xp, hist, lin_w, lin_b.reshape(1, C),
      _eff(tc1_w1), tc1_b1.reshape(1, C),
      _eff(tc1_w2), tc1_b2.reshape(1, C),
      _eff(tc1_w3), tc1_b3.reshape(1, C))
  p1 = _hop()(rowp, colp, v1)
  tx1, v2 = _tc2(p1, disb)
  p2 = _hop()(rowp, colp, v2)
  o = _tc3(
      p2, disb, g, tx1,
      cheb_w[0], cheb_w[1], cheb_w[2], cheb_b.reshape(1, C),
      _eff(tc2_w1), tc2_b1.reshape(1, C),
      _eff(tc2_w2), tc2_b2.reshape(1, C),
      _eff(tc2_w3), tc2_b3.reshape(1, C))
  return jnp.broadcast_to(o[:N, None, :], (N, T_OUT, C))


# speedup vs baseline: 270.7283x; 1.7124x over previous
import functools

import jax
import jax.numpy as jnp
from jax import lax
from jax.experimental import pallas as pl
from jax.experimental.pallas import tpu as pltpu
from jax.experimental.pallas import tpu_sc as plsc

N = 10000
F_IN = 128
C = 32
T_OUT = 8

N_PAD = 10240
PADV = N
E = 320000
E2 = 2 * E
NC, NS, L = 2, 16, 16
NW = NC * NS
CHUNK = 128
EDGES_PER_W = 20480
E_PAD = NW * EDGES_PER_W
NCHUNKS = EDGES_PER_W // CHUNK
ROWS_PER_TILE = N_PAD // NS

def _sc_mesh():
  return plsc.VectorSubcoreMesh(
      core_axis_name="c", subcore_axis_name="s", num_cores=NC, num_subcores=NS)


def _hist_body(idx2_hbm, out_hbm, idx2, ones_v, zbuf, acc_sh):
  c = lax.axis_index("c")
  s = lax.axis_index("s")
  wid = s * NC + c

  def fill_ones(i, _):
    ones_v[i, :] = jnp.full((L,), 1.0, jnp.float32)
    return 0
  lax.fori_loop(0, CHUNK, fill_ones, 0)

  def fill_z(i, _):
    zbuf[i, :] = jnp.zeros((L,), jnp.float32)
    return 0
  lax.fori_loop(0, ROWS_PER_TILE, fill_z, 0)
  pltpu.sync_copy(zbuf, acc_sh.at[pl.ds(s * ROWS_PER_TILE, ROWS_PER_TILE)])
  pltpu.sync_copy(idx2_hbm.at[pl.ds(wid * NCHUNKS, NCHUNKS)], idx2)
  plsc.subcore_barrier()

  def body(k, _):
    pltpu.sync_copy(ones_v, acc_sh.at[idx2.at[k]], add=True)
    return 0
  lax.fori_loop(0, NCHUNKS, body, 0)
  plsc.subcore_barrier()

  pltpu.sync_copy(
      acc_sh.at[pl.ds(s * ROWS_PER_TILE, ROWS_PER_TILE)],
      out_hbm.at[pl.ds(c * N_PAD + s * ROWS_PER_TILE, ROWS_PER_TILE)])


@functools.cache
def _hist():
  return pl.kernel(
      _hist_body,
      out_type=jax.ShapeDtypeStruct((NC * N_PAD, L), jnp.float32),
      mesh=_sc_mesh(),
      scratch_types=[
          pltpu.VMEM((NCHUNKS, CHUNK), jnp.int32),
          pltpu.VMEM((CHUNK, L), jnp.float32),
          pltpu.VMEM((ROWS_PER_TILE, L), jnp.float32),
          pltpu.VMEM_SHARED((N_PAD, L), jnp.float32),
      ],
      compiler_params=pltpu.CompilerParams(use_tc_tiling_on_sc=False),
  )


def _hop_body(row2_hbm, col2_hbm, tab_hbm, out_hbm,
              ridx2, cidx2, rows_v, zbuf, acc_sh, sem0, sem1):
  c = lax.axis_index("c")
  s = lax.axis_index("s")
  wid = s * NC + c

  def fill(i, _):
    zbuf[i, pl.ds(0, L)] = jnp.zeros((L,), jnp.float32)
    zbuf[i, pl.ds(L, L)] = jnp.zeros((L,), jnp.float32)
    return 0
  lax.fori_loop(0, ROWS_PER_TILE, fill, 0)
  pltpu.sync_copy(zbuf, acc_sh.at[pl.ds(s * ROWS_PER_TILE, ROWS_PER_TILE)])
  pltpu.sync_copy(row2_hbm.at[pl.ds(wid * NCHUNKS, NCHUNKS)], ridx2)
  pltpu.sync_copy(col2_hbm.at[pl.ds(wid * NCHUNKS, NCHUNKS)], cidx2)
  plsc.subcore_barrier()

  sems = (sem0, sem1)
  for b in range(2):
    pltpu.async_copy(tab_hbm.at[ridx2.at[b]], rows_v.at[b], sems[b])

  def outer(j, _):
    k0 = j * 2
    for b in range(2):
      k = k0 + b
      pltpu.make_async_copy(
          tab_hbm.at[pl.ds(0, CHUNK)], rows_v.at[b], sems[b]).wait()
      pltpu.sync_copy(rows_v.at[b], acc_sh.at[cidx2.at[k]], add=True)

      @pl.when(k + 2 < NCHUNKS)
      def _():
        pltpu.async_copy(tab_hbm.at[ridx2.at[k + 2]], rows_v.at[b], sems[b])
    return 0
  lax.fori_loop(0, NCHUNKS // 2, outer, 0)
  plsc.subcore_barrier()

  pltpu.sync_copy(
      acc_sh.at[pl.ds(s * ROWS_PER_TILE, ROWS_PER_TILE)],
      out_hbm.at[pl.ds(c * N_PAD + s * ROWS_PER_TILE, ROWS_PER_TILE)])


@functools.cache
def _hop():
  return pl.kernel(
      _hop_body,
      out_type=jax.ShapeDtypeStruct((NC * N_PAD, C), jnp.float32),
      mesh=_sc_mesh(),
      scratch_types=[
          pltpu.VMEM((NCHUNKS, CHUNK), jnp.int32),
          pltpu.VMEM((NCHUNKS, CHUNK), jnp.int32),
          pltpu.VMEM((2, CHUNK, C), jnp.float32),
          pltpu.VMEM((ROWS_PER_TILE, C), jnp.float32),
          pltpu.VMEM_SHARED((N_PAD, C), jnp.float32),
          pltpu.SemaphoreType.DMA,
          pltpu.SemaphoreType.DMA,
      ],
      compiler_params=pltpu.CompilerParams(use_tc_tiling_on_sc=False),
  )


def _sigmoid(x):
  return 1.0 / (1.0 + jnp.exp(-x))


def _tc1_body(x_ref, hist_ref, lw_ref, lb_ref, a1_ref, b1_ref, a2_ref,
              b2_ref, a3_ref, b3_ref, g_ref, v1_ref, disb_ref):
  x = x_ref[...]
  h = jnp.dot(x, lw_ref[...], preferred_element_type=jnp.float32) + lb_ref[...]
  p = jnp.dot(h, a1_ref[...], preferred_element_type=jnp.float32) + b1_ref[...]
  q = jnp.dot(h, a2_ref[...], preferred_element_type=jnp.float32) + b2_ref[...]
  r = jnp.dot(h, a3_ref[...], preferred_element_type=jnp.float32) + b3_ref[...]
  g = jnp.maximum(p * _sigmoid(q) + r, 0.0)

  hist = hist_ref[...]
  deg16 = hist[:N_PAD, :] + hist[N_PAD:, :]
  rowid16 = lax.broadcasted_iota(jnp.int32, (N_PAD, L), 0)
  dis16 = jnp.where((deg16 > 0.0) & (rowid16 < N), lax.rsqrt(deg16), 0.0)
  disb = jnp.concatenate([dis16, dis16], axis=1)

  rowid = lax.broadcasted_iota(jnp.int32, (N_PAD, C), 0)
  g = jnp.where(rowid < N, g, 0.0)
  g_ref[...] = g
  v1_ref[...] = disb * g
  disb_ref[...] = disb


def _tc2_body(p_ref, disb_ref, tx1_ref, v2_ref):
  p = p_ref[...]
  disb = disb_ref[...]
  u1 = p[:N_PAD, :] + p[N_PAD:, :]
  tx1 = -disb * u1
  tx1_ref[...] = tx1
  v2_ref[...] = disb * tx1


def _tc3_body(q_ref, disb_ref, g_ref, tx1_ref, w0_ref, w1_ref, w2_ref,
              cb_ref, d1_ref, e1_ref, d2_ref, e2_ref, d3_ref, e3_ref, o_ref):
  q = q_ref[...]
  disb = disb_ref[...]
  g = g_ref[...]
  tx1 = tx1_ref[...]
  u2 = q[:N_PAD, :] + q[N_PAD:, :]
  tx2 = -2.0 * disb * u2 - g
  cheb = (jnp.dot(g, w0_ref[...], preferred_element_type=jnp.float32)
          + jnp.dot(tx1, w1_ref[...], preferred_element_type=jnp.float32)
          + jnp.dot(tx2, w2_ref[...], preferred_element_type=jnp.float32)
          + cb_ref[...])
  cheb = jnp.maximum(cheb, 0.0)
  p2 = jnp.dot(cheb, d1_ref[...], preferred_element_type=jnp.float32) + e1_ref[...]
  q2 = jnp.dot(cheb, d2_ref[...], preferred_element_type=jnp.float32) + e2_ref[...]
  r2 = jnp.dot(cheb, d3_ref[...], preferred_element_type=jnp.float32) + e3_ref[...]
  o_ref[...] = jnp.maximum(p2 * _sigmoid(q2) + r2, 0.0)


_f32 = lambda *shape: jax.ShapeDtypeStruct(shape, jnp.float32)

_tc1 = pl.pallas_call(
    _tc1_body, out_shape=[_f32(N_PAD, C), _f32(N_PAD, C), _f32(N_PAD, C)])
_tc2 = pl.pallas_call(_tc2_body, out_shape=[_f32(N_PAD, C), _f32(N_PAD, C)])
_tc3 = pl.pallas_call(_tc3_body, out_shape=_f32(N_PAD, C))


def _eff(w):
  return jnp.transpose(jnp.sum(w[:, :, 0, :], axis=-1), (1, 0))


def kernel(x, edge_index, lin_w, lin_b, tc1_w1, tc1_b1, tc1_w2, tc1_b2,
           tc1_w3, tc1_b3, cheb_w, cheb_b, tc2_w1, tc2_b1, tc2_w2, tc2_b2,
           tc2_w3, tc2_b3):
  xp = jnp.pad(x, ((0, N_PAD - N), (0, 0)))
  ei0 = edge_index[0]
  ei1 = edge_index[1]
  pad = jnp.full((E_PAD - E2,), PADV, jnp.int32)
  rowp = jnp.concatenate([ei0, ei1, pad]).reshape(NW * NCHUNKS, CHUNK)
  colp = jnp.concatenate([ei1, ei0, pad]).reshape(NW * NCHUNKS, CHUNK)

  hist = _hist()(rowp)
  g, v1, disb = _tc1(
      xp, hist, lin_w, lin_b.reshape(1, C),
      _eff(tc1_w1), tc1_b1.reshape(1, C),
      _eff(tc1_w2), tc1_b2.reshape(1, C),
      _eff(tc1_w3), tc1_b3.reshape(1, C))
  p1 = _hop()(rowp, colp, v1)
  tx1, v2 = _tc2(p1, disb)
  p2 = _hop()(rowp, colp, v2)
  o = _tc3(
      p2, disb, g, tx1,
      cheb_w[0], cheb_w[1], cheb_w[2], cheb_b.reshape(1, C),
      _eff(tc2_w1), tc2_b1.reshape(1, C),
      _eff(tc2_w2), tc2_b2.reshape(1, C),
      _eff(tc2_w3), tc2_b3.reshape(1, C))
  return jnp.broadcast_to(o[:N, None, :], (N, T_OUT, C))


# speedup vs baseline: 275.6534x; 1.0182x over previous
import functools

import jax
import jax.numpy as jnp
from jax import lax
from jax.experimental import pallas as pl
from jax.experimental.pallas import tpu as pltpu
from jax.experimental.pallas import tpu_sc as plsc

N = 10000
F_IN = 128
C = 32
T_OUT = 8

N_PAD = 10240
PADV = N
E = 320000
E2 = 2 * E
NC, NS, L = 2, 16, 16
NW = NC * NS
CHUNK = 128
EDGES_PER_W = 20480
E_PAD = NW * EDGES_PER_W
NCHUNKS = EDGES_PER_W // CHUNK
ROWS_PER_TILE = N_PAD // NS

def _sc_mesh():
  return plsc.VectorSubcoreMesh(
      core_axis_name="c", subcore_axis_name="s", num_cores=NC, num_subcores=NS)


def _hist_body(idx2_hbm, out_hbm, idx2, ones_v, zbuf, acc_sh):
  c = lax.axis_index("c")
  s = lax.axis_index("s")
  wid = s * NC + c

  def fill_ones(i, _):
    ones_v[i, :] = jnp.full((L,), 1.0, jnp.float32)
    return 0
  lax.fori_loop(0, CHUNK, fill_ones, 0)

  def fill_z(i, _):
    zbuf[i, :] = jnp.zeros((L,), jnp.float32)
    return 0
  lax.fori_loop(0, ROWS_PER_TILE, fill_z, 0)
  pltpu.sync_copy(zbuf, acc_sh.at[pl.ds(s * ROWS_PER_TILE, ROWS_PER_TILE)])
  pltpu.sync_copy(idx2_hbm.at[pl.ds(wid * NCHUNKS, NCHUNKS)], idx2)
  plsc.subcore_barrier()

  def body(k, _):
    pltpu.sync_copy(ones_v, acc_sh.at[idx2.at[k]], add=True)
    return 0
  lax.fori_loop(0, NCHUNKS, body, 0)
  plsc.subcore_barrier()

  pltpu.sync_copy(
      acc_sh.at[pl.ds(s * ROWS_PER_TILE, ROWS_PER_TILE)],
      out_hbm.at[pl.ds(c * N_PAD + s * ROWS_PER_TILE, ROWS_PER_TILE)])


@functools.cache
def _hist():
  return pl.kernel(
      _hist_body,
      out_type=jax.ShapeDtypeStruct((NC * N_PAD, L), jnp.float32),
      mesh=_sc_mesh(),
      scratch_types=[
          pltpu.VMEM((NCHUNKS, CHUNK), jnp.int32),
          pltpu.VMEM((CHUNK, L), jnp.float32),
          pltpu.VMEM((ROWS_PER_TILE, L), jnp.float32),
          pltpu.VMEM_SHARED((N_PAD, L), jnp.float32),
      ],
      compiler_params=pltpu.CompilerParams(use_tc_tiling_on_sc=False),
  )


NBUF = 4


def _hop_body(row2_hbm, col2_hbm, tab_hbm, out_hbm,
              ridx2, cidx2, rows_v, zbuf, acc_sh, *sems):
  c = lax.axis_index("c")
  s = lax.axis_index("s")
  wid = s * NC + c

  def fill(i, _):
    zbuf[i, pl.ds(0, L)] = jnp.zeros((L,), jnp.float32)
    zbuf[i, pl.ds(L, L)] = jnp.zeros((L,), jnp.float32)
    return 0
  lax.fori_loop(0, ROWS_PER_TILE, fill, 0)
  pltpu.sync_copy(zbuf, acc_sh.at[pl.ds(s * ROWS_PER_TILE, ROWS_PER_TILE)])
  pltpu.sync_copy(row2_hbm.at[pl.ds(wid * NCHUNKS, NCHUNKS)], ridx2)
  pltpu.sync_copy(col2_hbm.at[pl.ds(wid * NCHUNKS, NCHUNKS)], cidx2)
  plsc.subcore_barrier()

  for b in range(NBUF):
    pltpu.async_copy(tab_hbm.at[ridx2.at[b]], rows_v.at[b], sems[b])

  def outer(j, _):
    k0 = j * NBUF
    for b in range(NBUF):
      k = k0 + b
      pltpu.make_async_copy(
          tab_hbm.at[pl.ds(0, CHUNK)], rows_v.at[b], sems[b]).wait()
      pltpu.sync_copy(rows_v.at[b], acc_sh.at[cidx2.at[k]], add=True)

      @pl.when(k + NBUF < NCHUNKS)
      def _():
        pltpu.async_copy(tab_hbm.at[ridx2.at[k + NBUF]], rows_v.at[b], sems[b])
    return 0
  lax.fori_loop(0, NCHUNKS // NBUF, outer, 0)
  plsc.subcore_barrier()

  pltpu.sync_copy(
      acc_sh.at[pl.ds(s * ROWS_PER_TILE, ROWS_PER_TILE)],
      out_hbm.at[pl.ds(c * N_PAD + s * ROWS_PER_TILE, ROWS_PER_TILE)])


@functools.cache
def _hop():
  return pl.kernel(
      _hop_body,
      out_type=jax.ShapeDtypeStruct((NC * N_PAD, C), jnp.float32),
      mesh=_sc_mesh(),
      scratch_types=[
          pltpu.VMEM((NCHUNKS, CHUNK), jnp.int32),
          pltpu.VMEM((NCHUNKS, CHUNK), jnp.int32),
          pltpu.VMEM((NBUF, CHUNK, C), jnp.float32),
          pltpu.VMEM((ROWS_PER_TILE, C), jnp.float32),
          pltpu.VMEM_SHARED((N_PAD, C), jnp.float32),
      ] + [pltpu.SemaphoreType.DMA] * NBUF,
      compiler_params=pltpu.CompilerParams(use_tc_tiling_on_sc=False),
  )


def _sigmoid(x):
  return 1.0 / (1.0 + jnp.exp(-x))


def _tc1_body(x_ref, hist_ref, lw_ref, lb_ref, a1_ref, b1_ref, a2_ref,
              b2_ref, a3_ref, b3_ref, g_ref, v1_ref, disb_ref):
  x = x_ref[...]
  h = jnp.dot(x, lw_ref[...], preferred_element_type=jnp.float32) + lb_ref[...]
  p = jnp.dot(h, a1_ref[...], preferred_element_type=jnp.float32) + b1_ref[...]
  q = jnp.dot(h, a2_ref[...], preferred_element_type=jnp.float32) + b2_ref[...]
  r = jnp.dot(h, a3_ref[...], preferred_element_type=jnp.float32) + b3_ref[...]
  g = jnp.maximum(p * _sigmoid(q) + r, 0.0)

  hist = hist_ref[...]
  deg16 = hist[:N_PAD, :] + hist[N_PAD:, :]
  rowid16 = lax.broadcasted_iota(jnp.int32, (N_PAD, L), 0)
  dis16 = jnp.where((deg16 > 0.0) & (rowid16 < N), lax.rsqrt(deg16), 0.0)
  disb = jnp.concatenate([dis16, dis16], axis=1)

  rowid = lax.broadcasted_iota(jnp.int32, (N_PAD, C), 0)
  g = jnp.where(rowid < N, g, 0.0)
  g_ref[...] = g
  v1_ref[...] = disb * g
  disb_ref[...] = disb


def _tc2_body(p_ref, disb_ref, tx1_ref, v2_ref):
  p = p_ref[...]
  disb = disb_ref[...]
  u1 = p[:N_PAD, :] + p[N_PAD:, :]
  tx1 = -disb * u1
  tx1_ref[...] = tx1
  v2_ref[...] = disb * tx1


def _tc3_body(q_ref, disb_ref, g_ref, tx1_ref, w0_ref, w1_ref, w2_ref,
              cb_ref, d1_ref, e1_ref, d2_ref, e2_ref, d3_ref, e3_ref, o_ref):
  q = q_ref[...]
  disb = disb_ref[...]
  g = g_ref[...]
  tx1 = tx1_ref[...]
  u2 = q[:N_PAD, :] + q[N_PAD:, :]
  tx2 = -2.0 * disb * u2 - g
  cheb = (jnp.dot(g, w0_ref[...], preferred_element_type=jnp.float32)
          + jnp.dot(tx1, w1_ref[...], preferred_element_type=jnp.float32)
          + jnp.dot(tx2, w2_ref[...], preferred_element_type=jnp.float32)
          + cb_ref[...])
  cheb = jnp.maximum(cheb, 0.0)
  p2 = jnp.dot(cheb, d1_ref[...], preferred_element_type=jnp.float32) + e1_ref[...]
  q2 = jnp.dot(cheb, d2_ref[...], preferred_element_type=jnp.float32) + e2_ref[...]
  r2 = jnp.dot(cheb, d3_ref[...], preferred_element_type=jnp.float32) + e3_ref[...]
  o_ref[...] = jnp.maximum(p2 * _sigmoid(q2) + r2, 0.0)


_f32 = lambda *shape: jax.ShapeDtypeStruct(shape, jnp.float32)

_tc1 = pl.pallas_call(
    _tc1_body, out_shape=[_f32(N_PAD, C), _f32(N_PAD, C), _f32(N_PAD, C)])
_tc2 = pl.pallas_call(_tc2_body, out_shape=[_f32(N_PAD, C), _f32(N_PAD, C)])
_tc3 = pl.pallas_call(_tc3_body, out_shape=_f32(N_PAD, C))


def _eff(w):
  return jnp.transpose(jnp.sum(w[:, :, 0, :], axis=-1), (1, 0))


def kernel(x, edge_index, lin_w, lin_b, tc1_w1, tc1_b1, tc1_w2, tc1_b2,
           tc1_w3, tc1_b3, cheb_w, cheb_b, tc2_w1, tc2_b1, tc2_w2, tc2_b2,
           tc2_w3, tc2_b3):
  xp = jnp.pad(x, ((0, N_PAD - N), (0, 0)))
  ei0 = edge_index[0]
  ei1 = edge_index[1]
  pad = jnp.full((E_PAD - E2,), PADV, jnp.int32)
  rowp = jnp.concatenate([ei0, ei1, pad]).reshape(NW * NCHUNKS, CHUNK)
  colp = jnp.concatenate([ei1, ei0, pad]).reshape(NW * NCHUNKS, CHUNK)

  hist = _hist()(rowp)
  g, v1, disb = _tc1(
      xp, hist, lin_w, lin_b.reshape(1, C),
      _eff(tc1_w1), tc1_b1.reshape(1, C),
      _eff(tc1_w2), tc1_b2.reshape(1, C),
      _eff(tc1_w3), tc1_b3.reshape(1, C))
  p1 = _hop()(rowp, colp, v1)
  tx1, v2 = _tc2(p1, disb)
  p2 = _hop()(rowp, colp, v2)
  o = _tc3(
      p2, disb, g, tx1,
      cheb_w[0], cheb_w[1], cheb_w[2], cheb_b.reshape(1, C),
      _eff(tc2_w1), tc2_b1.reshape(1, C),
      _eff(tc2_w2), tc2_b2.reshape(1, C),
      _eff(tc2_w3), tc2_b3.reshape(1, C))
  return jnp.broadcast_to(o[:N, None, :], (N, T_OUT, C))


# speedup vs baseline: 632.0602x; 2.2930x over previous
import functools

import jax
import jax.numpy as jnp
from jax import lax
from jax.experimental import pallas as pl
from jax.experimental.pallas import tpu as pltpu
from jax.experimental.pallas import tpu_sc as plsc

N = 10000
F_IN = 128
C = 32
T_OUT = 8

N_PAD = 10240
PADV = N
E = 320000
E2 = 2 * E
NC, NS, L = 2, 16, 16
NW = NC * NS
CHUNK = 128
EDGES_PER_W = 20480
E_PAD = NW * EDGES_PER_W
NCHUNKS = EDGES_PER_W // CHUNK
ROWS_PER_TILE = N_PAD // NS

def _sc_mesh():
  return plsc.VectorSubcoreMesh(
      core_axis_name="c", subcore_axis_name="s", num_cores=NC, num_subcores=NS)


def _hist_body(idx2_hbm, out_hbm, idx2, ones_v, zbuf, acc_sh):
  c = lax.axis_index("c")
  s = lax.axis_index("s")
  wid = s * NC + c

  def fill_ones(i, _):
    ones_v[i, :] = jnp.full((L,), 1.0, jnp.float32)
    return 0
  lax.fori_loop(0, CHUNK, fill_ones, 0)

  def fill_z(i, _):
    zbuf[i, :] = jnp.zeros((L,), jnp.float32)
    return 0
  lax.fori_loop(0, ROWS_PER_TILE, fill_z, 0)
  pltpu.sync_copy(zbuf, acc_sh.at[pl.ds(s * ROWS_PER_TILE, ROWS_PER_TILE)])
  pltpu.sync_copy(idx2_hbm.at[pl.ds(wid * NCHUNKS, NCHUNKS)], idx2)
  plsc.subcore_barrier()

  def body(k, _):
    pltpu.sync_copy(ones_v, acc_sh.at[idx2.at[k]], add=True)
    return 0
  lax.fori_loop(0, NCHUNKS, body, 0)
  plsc.subcore_barrier()

  pltpu.sync_copy(
      acc_sh.at[pl.ds(s * ROWS_PER_TILE, ROWS_PER_TILE)],
      out_hbm.at[pl.ds(c * N_PAD + s * ROWS_PER_TILE, ROWS_PER_TILE)])


@functools.cache
def _hist():
  return pl.kernel(
      _hist_body,
      out_type=jax.ShapeDtypeStruct((NC * N_PAD, L), jnp.float32),
      mesh=_sc_mesh(),
      scratch_types=[
          pltpu.VMEM((NCHUNKS, CHUNK), jnp.int32),
          pltpu.VMEM((CHUNK, L), jnp.float32),
          pltpu.VMEM((ROWS_PER_TILE, L), jnp.float32),
          pltpu.VMEM_SHARED((N_PAD, L), jnp.float32),
      ],
      compiler_params=pltpu.CompilerParams(use_tc_tiling_on_sc=False),
  )


NBUF = 4


def _hop_body(row2_hbm, col2_hbm, tab_hbm, out_hbm,
              ridx2, cidx2, rows_v, zbuf, acc_sh, tab_sh, *sems):
  c = lax.axis_index("c")
  s = lax.axis_index("s")
  wid = s * NC + c

  def fill(i, _):
    zbuf[i, pl.ds(0, L)] = jnp.zeros((L,), jnp.float32)
    zbuf[i, pl.ds(L, L)] = jnp.zeros((L,), jnp.float32)
    return 0
  lax.fori_loop(0, ROWS_PER_TILE, fill, 0)
  pltpu.sync_copy(zbuf, acc_sh.at[pl.ds(s * ROWS_PER_TILE, ROWS_PER_TILE)])
  pltpu.sync_copy(tab_hbm.at[pl.ds(s * ROWS_PER_TILE, ROWS_PER_TILE)],
                  tab_sh.at[pl.ds(s * ROWS_PER_TILE, ROWS_PER_TILE)])
  pltpu.sync_copy(row2_hbm.at[pl.ds(wid * NCHUNKS, NCHUNKS)], ridx2)
  pltpu.sync_copy(col2_hbm.at[pl.ds(wid * NCHUNKS, NCHUNKS)], cidx2)
  plsc.subcore_barrier()

  for b in range(NBUF):
    pltpu.async_copy(tab_sh.at[ridx2.at[b]], rows_v.at[b], sems[b])

  def outer(j, _):
    k0 = j * NBUF
    for b in range(NBUF):
      k = k0 + b
      pltpu.make_async_copy(
          tab_hbm.at[pl.ds(0, CHUNK)], rows_v.at[b], sems[b]).wait()
      pltpu.sync_copy(rows_v.at[b], acc_sh.at[cidx2.at[k]], add=True)

      @pl.when(k + NBUF < NCHUNKS)
      def _():
        pltpu.async_copy(tab_sh.at[ridx2.at[k + NBUF]], rows_v.at[b], sems[b])
    return 0
  lax.fori_loop(0, NCHUNKS // NBUF, outer, 0)
  plsc.subcore_barrier()

  pltpu.sync_copy(
      acc_sh.at[pl.ds(s * ROWS_PER_TILE, ROWS_PER_TILE)],
      out_hbm.at[pl.ds(c * N_PAD + s * ROWS_PER_TILE, ROWS_PER_TILE)])


@functools.cache
def _hop():
  return pl.kernel(
      _hop_body,
      out_type=jax.ShapeDtypeStruct((NC * N_PAD, C), jnp.float32),
      mesh=_sc_mesh(),
      scratch_types=[
          pltpu.VMEM((NCHUNKS, CHUNK), jnp.int32),
          pltpu.VMEM((NCHUNKS, CHUNK), jnp.int32),
          pltpu.VMEM((NBUF, CHUNK, C), jnp.float32),
          pltpu.VMEM((ROWS_PER_TILE, C), jnp.float32),
          pltpu.VMEM_SHARED((N_PAD, C), jnp.float32),
          pltpu.VMEM_SHARED((N_PAD, C), jnp.float32),
      ] + [pltpu.SemaphoreType.DMA] * NBUF,
      compiler_params=pltpu.CompilerParams(use_tc_tiling_on_sc=False),
  )


def _sigmoid(x):
  return 1.0 / (1.0 + jnp.exp(-x))


def _tc1_body(x_ref, hist_ref, lw_ref, lb_ref, a1_ref, b1_ref, a2_ref,
              b2_ref, a3_ref, b3_ref, g_ref, v1_ref, disb_ref):
  x = x_ref[...]
  h = jnp.dot(x, lw_ref[...], preferred_element_type=jnp.float32) + lb_ref[...]
  p = jnp.dot(h, a1_ref[...], preferred_element_type=jnp.float32) + b1_ref[...]
  q = jnp.dot(h, a2_ref[...], preferred_element_type=jnp.float32) + b2_ref[...]
  r = jnp.dot(h, a3_ref[...], preferred_element_type=jnp.float32) + b3_ref[...]
  g = jnp.maximum(p * _sigmoid(q) + r, 0.0)

  hist = hist_ref[...]
  deg16 = hist[:N_PAD, :] + hist[N_PAD:, :]
  rowid16 = lax.broadcasted_iota(jnp.int32, (N_PAD, L), 0)
  dis16 = jnp.where((deg16 > 0.0) & (rowid16 < N), lax.rsqrt(deg16), 0.0)
  disb = jnp.concatenate([dis16, dis16], axis=1)

  rowid = lax.broadcasted_iota(jnp.int32, (N_PAD, C), 0)
  g = jnp.where(rowid < N, g, 0.0)
  g_ref[...] = g
  v1_ref[...] = disb * g
  disb_ref[...] = disb


def _tc2_body(p_ref, disb_ref, tx1_ref, v2_ref):
  p = p_ref[...]
  disb = disb_ref[...]
  u1 = p[:N_PAD, :] + p[N_PAD:, :]
  tx1 = -disb * u1
  tx1_ref[...] = tx1
  v2_ref[...] = disb * tx1


def _tc3_body(q_ref, disb_ref, g_ref, tx1_ref, w0_ref, w1_ref, w2_ref,
              cb_ref, d1_ref, e1_ref, d2_ref, e2_ref, d3_ref, e3_ref, o_ref):
  q = q_ref[...]
  disb = disb_ref[...]
  g = g_ref[...]
  tx1 = tx1_ref[...]
  u2 = q[:N_PAD, :] + q[N_PAD:, :]
  tx2 = -2.0 * disb * u2 - g
  cheb = (jnp.dot(g, w0_ref[...], preferred_element_type=jnp.float32)
          + jnp.dot(tx1, w1_ref[...], preferred_element_type=jnp.float32)
          + jnp.dot(tx2, w2_ref[...], preferred_element_type=jnp.float32)
          + cb_ref[...])
  cheb = jnp.maximum(cheb, 0.0)
  p2 = jnp.dot(cheb, d1_ref[...], preferred_element_type=jnp.float32) + e1_ref[...]
  q2 = jnp.dot(cheb, d2_ref[...], preferred_element_type=jnp.float32) + e2_ref[...]
  r2 = jnp.dot(cheb, d3_ref[...], preferred_element_type=jnp.float32) + e3_ref[...]
  o_ref[...] = jnp.maximum(p2 * _sigmoid(q2) + r2, 0.0)


_f32 = lambda *shape: jax.ShapeDtypeStruct(shape, jnp.float32)

_tc1 = pl.pallas_call(
    _tc1_body, out_shape=[_f32(N_PAD, C), _f32(N_PAD, C), _f32(N_PAD, C)])
_tc2 = pl.pallas_call(_tc2_body, out_shape=[_f32(N_PAD, C), _f32(N_PAD, C)])
_tc3 = pl.pallas_call(_tc3_body, out_shape=_f32(N_PAD, C))


def _eff(w):
  return jnp.transpose(jnp.sum(w[:, :, 0, :], axis=-1), (1, 0))


def kernel(x, edge_index, lin_w, lin_b, tc1_w1, tc1_b1, tc1_w2, tc1_b2,
           tc1_w3, tc1_b3, cheb_w, cheb_b, tc2_w1, tc2_b1, tc2_w2, tc2_b2,
           tc2_w3, tc2_b3):
  xp = jnp.pad(x, ((0, N_PAD - N), (0, 0)))
  ei0 = edge_index[0]
  ei1 = edge_index[1]
  pad = jnp.full((E_PAD - E2,), PADV, jnp.int32)
  rowp = jnp.concatenate([ei0, ei1, pad]).reshape(NW * NCHUNKS, CHUNK)
  colp = jnp.concatenate([ei1, ei0, pad]).reshape(NW * NCHUNKS, CHUNK)

  hist = _hist()(rowp)
  g, v1, disb = _tc1(
      xp, hist, lin_w, lin_b.reshape(1, C),
      _eff(tc1_w1), tc1_b1.reshape(1, C),
      _eff(tc1_w2), tc1_b2.reshape(1, C),
      _eff(tc1_w3), tc1_b3.reshape(1, C))
  p1 = _hop()(rowp, colp, v1)
  tx1, v2 = _tc2(p1, disb)
  p2 = _hop()(rowp, colp, v2)
  o = _tc3(
      p2, disb, g, tx1,
      cheb_w[0], cheb_w[1], cheb_w[2], cheb_b.reshape(1, C),
      _eff(tc2_w1), tc2_b1.reshape(1, C),
      _eff(tc2_w2), tc2_b2.reshape(1, C),
      _eff(tc2_w3), tc2_b3.reshape(1, C))
  return jnp.broadcast_to(o[:N, None, :], (N, T_OUT, C))


# speedup vs baseline: 639.7298x; 1.0121x over previous
import functools

import jax
import jax.numpy as jnp
from jax import lax
from jax.experimental import pallas as pl
from jax.experimental.pallas import tpu as pltpu
from jax.experimental.pallas import tpu_sc as plsc

N = 10000
F_IN = 128
C = 32
T_OUT = 8

N_PAD = 10240
PADV = N
E = 320000
E2 = 2 * E
NC, NS, L = 2, 16, 16
NW = NC * NS
CHUNK = 128
EDGES_PER_W = 20480
E_PAD = NW * EDGES_PER_W
NCHUNKS = EDGES_PER_W // CHUNK
ROWS_PER_TILE = N_PAD // NS
ZROWS = 64

def _sc_mesh():
  return plsc.VectorSubcoreMesh(
      core_axis_name="c", subcore_axis_name="s", num_cores=NC, num_subcores=NS)


def _hist_body(idx2_hbm, out_hbm, idx2, ones_v, zbuf, acc_sh):
  c = lax.axis_index("c")
  s = lax.axis_index("s")
  wid = s * NC + c

  def fill_ones(i, _):
    ones_v[i, :] = jnp.full((L,), 1.0, jnp.float32)
    return 0
  lax.fori_loop(0, CHUNK, fill_ones, 0)

  def fill_z(i, _):
    zbuf[i, :] = jnp.zeros((L,), jnp.float32)
    return 0
  lax.fori_loop(0, ZROWS, fill_z, 0)

  def zero_slice(t, _):
    pltpu.sync_copy(
        zbuf, acc_sh.at[pl.ds(s * ROWS_PER_TILE + t * ZROWS, ZROWS)])
    return 0
  lax.fori_loop(0, ROWS_PER_TILE // ZROWS, zero_slice, 0)
  pltpu.sync_copy(idx2_hbm.at[pl.ds(wid * NCHUNKS, NCHUNKS)], idx2)
  plsc.subcore_barrier()

  def body(k, _):
    pltpu.sync_copy(ones_v, acc_sh.at[idx2.at[k]], add=True)
    return 0
  lax.fori_loop(0, NCHUNKS, body, 0)
  plsc.subcore_barrier()

  pltpu.sync_copy(
      acc_sh.at[pl.ds(s * ROWS_PER_TILE, ROWS_PER_TILE)],
      out_hbm.at[pl.ds(c * N_PAD + s * ROWS_PER_TILE, ROWS_PER_TILE)])


@functools.cache
def _hist():
  return pl.kernel(
      _hist_body,
      out_type=jax.ShapeDtypeStruct((NC * N_PAD, L), jnp.float32),
      mesh=_sc_mesh(),
      scratch_types=[
          pltpu.VMEM((NCHUNKS, CHUNK), jnp.int32),
          pltpu.VMEM((CHUNK, L), jnp.float32),
          pltpu.VMEM((ZROWS, L), jnp.float32),
          pltpu.VMEM_SHARED((N_PAD, L), jnp.float32),
      ],
      compiler_params=pltpu.CompilerParams(use_tc_tiling_on_sc=False),
  )


NBUF = 8


def _hop_body(row2_hbm, col2_hbm, tab_hbm, out_hbm,
              ridx2, cidx2, rows_v, zbuf, acc_sh, tab_sh, *sems):
  c = lax.axis_index("c")
  s = lax.axis_index("s")
  wid = s * NC + c

  def fill(i, _):
    zbuf[i, pl.ds(0, L)] = jnp.zeros((L,), jnp.float32)
    zbuf[i, pl.ds(L, L)] = jnp.zeros((L,), jnp.float32)
    return 0
  lax.fori_loop(0, ZROWS, fill, 0)

  def zero_slice(t, _):
    pltpu.sync_copy(
        zbuf, acc_sh.at[pl.ds(s * ROWS_PER_TILE + t * ZROWS, ZROWS)])
    return 0
  lax.fori_loop(0, ROWS_PER_TILE // ZROWS, zero_slice, 0)
  pltpu.sync_copy(tab_hbm.at[pl.ds(s * ROWS_PER_TILE, ROWS_PER_TILE)],
                  tab_sh.at[pl.ds(s * ROWS_PER_TILE, ROWS_PER_TILE)])
  pltpu.sync_copy(row2_hbm.at[pl.ds(wid * NCHUNKS, NCHUNKS)], ridx2)
  pltpu.sync_copy(col2_hbm.at[pl.ds(wid * NCHUNKS, NCHUNKS)], cidx2)
  plsc.subcore_barrier()

  for b in range(NBUF):
    pltpu.async_copy(tab_sh.at[ridx2.at[b]], rows_v.at[b], sems[b])

  def outer(j, _):
    k0 = j * NBUF
    for b in range(NBUF):
      k = k0 + b
      pltpu.make_async_copy(
          tab_hbm.at[pl.ds(0, CHUNK)], rows_v.at[b], sems[b]).wait()
      pltpu.sync_copy(rows_v.at[b], acc_sh.at[cidx2.at[k]], add=True)

      @pl.when(k + NBUF < NCHUNKS)
      def _():
        pltpu.async_copy(tab_sh.at[ridx2.at[k + NBUF]], rows_v.at[b], sems[b])
    return 0
  lax.fori_loop(0, NCHUNKS // NBUF, outer, 0)
  plsc.subcore_barrier()

  pltpu.sync_copy(
      acc_sh.at[pl.ds(s * ROWS_PER_TILE, ROWS_PER_TILE)],
      out_hbm.at[pl.ds(c * N_PAD + s * ROWS_PER_TILE, ROWS_PER_TILE)])


@functools.cache
def _hop():
  return pl.kernel(
      _hop_body,
      out_type=jax.ShapeDtypeStruct((NC * N_PAD, C), jnp.float32),
      mesh=_sc_mesh(),
      scratch_types=[
          pltpu.VMEM((NCHUNKS, CHUNK), jnp.int32),
          pltpu.VMEM((NCHUNKS, CHUNK), jnp.int32),
          pltpu.VMEM((NBUF, CHUNK, C), jnp.float32),
          pltpu.VMEM((ZROWS, C), jnp.float32),
          pltpu.VMEM_SHARED((N_PAD, C), jnp.float32),
          pltpu.VMEM_SHARED((N_PAD, C), jnp.float32),
      ] + [pltpu.SemaphoreType.DMA] * NBUF,
      compiler_params=pltpu.CompilerParams(use_tc_tiling_on_sc=False),
  )


def _sigmoid(x):
  return 1.0 / (1.0 + jnp.exp(-x))


def _tc1_body(x_ref, hist_ref, lw_ref, lb_ref, a1_ref, b1_ref, a2_ref,
              b2_ref, a3_ref, b3_ref, g_ref, v1_ref, disb_ref):
  x = x_ref[...]
  h = jnp.dot(x, lw_ref[...], preferred_element_type=jnp.float32) + lb_ref[...]
  p = jnp.dot(h, a1_ref[...], preferred_element_type=jnp.float32) + b1_ref[...]
  q = jnp.dot(h, a2_ref[...], preferred_element_type=jnp.float32) + b2_ref[...]
  r = jnp.dot(h, a3_ref[...], preferred_element_type=jnp.float32) + b3_ref[...]
  g = jnp.maximum(p * _sigmoid(q) + r, 0.0)

  hist = hist_ref[...]
  deg16 = hist[:N_PAD, :] + hist[N_PAD:, :]
  rowid16 = lax.broadcasted_iota(jnp.int32, (N_PAD, L), 0)
  dis16 = jnp.where((deg16 > 0.0) & (rowid16 < N), lax.rsqrt(deg16), 0.0)
  disb = jnp.concatenate([dis16, dis16], axis=1)

  rowid = lax.broadcasted_iota(jnp.int32, (N_PAD, C), 0)
  g = jnp.where(rowid < N, g, 0.0)
  g_ref[...] = g
  v1_ref[...] = disb * g
  disb_ref[...] = disb


def _tc2_body(p_ref, disb_ref, tx1_ref, v2_ref):
  p = p_ref[...]
  disb = disb_ref[...]
  u1 = p[:N_PAD, :] + p[N_PAD:, :]
  tx1 = -disb * u1
  tx1_ref[...] = tx1
  v2_ref[...] = disb * tx1


def _tc3_body(q_ref, disb_ref, g_ref, tx1_ref, w0_ref, w1_ref, w2_ref,
              cb_ref, d1_ref, e1_ref, d2_ref, e2_ref, d3_ref, e3_ref, o_ref):
  q = q_ref[...]
  disb = disb_ref[...]
  g = g_ref[...]
  tx1 = tx1_ref[...]
  u2 = q[:N_PAD, :] + q[N_PAD:, :]
  tx2 = -2.0 * disb * u2 - g
  cheb = (jnp.dot(g, w0_ref[...], preferred_element_type=jnp.float32)
          + jnp.dot(tx1, w1_ref[...], preferred_element_type=jnp.float32)
          + jnp.dot(tx2, w2_ref[...], preferred_element_type=jnp.float32)
          + cb_ref[...])
  cheb = jnp.maximum(cheb, 0.0)
  p2 = jnp.dot(cheb, d1_ref[...], preferred_element_type=jnp.float32) + e1_ref[...]
  q2 = jnp.dot(cheb, d2_ref[...], preferred_element_type=jnp.float32) + e2_ref[...]
  r2 = jnp.dot(cheb, d3_ref[...], preferred_element_type=jnp.float32) + e3_ref[...]
  o_ref[...] = jnp.maximum(p2 * _sigmoid(q2) + r2, 0.0)


_f32 = lambda *shape: jax.ShapeDtypeStruct(shape, jnp.float32)

_tc1 = pl.pallas_call(
    _tc1_body, out_shape=[_f32(N_PAD, C), _f32(N_PAD, C), _f32(N_PAD, C)])
_tc2 = pl.pallas_call(_tc2_body, out_shape=[_f32(N_PAD, C), _f32(N_PAD, C)])
_tc3 = pl.pallas_call(_tc3_body, out_shape=_f32(N_PAD, C))


def _eff(w):
  return jnp.transpose(jnp.sum(w[:, :, 0, :], axis=-1), (1, 0))


def kernel(x, edge_index, lin_w, lin_b, tc1_w1, tc1_b1, tc1_w2, tc1_b2,
           tc1_w3, tc1_b3, cheb_w, cheb_b, tc2_w1, tc2_b1, tc2_w2, tc2_b2,
           tc2_w3, tc2_b3):
  xp = jnp.pad(x, ((0, N_PAD - N), (0, 0)))
  ei0 = edge_index[0]
  ei1 = edge_index[1]
  pad = jnp.full((E_PAD - E2,), PADV, jnp.int32)
  rowp = jnp.concatenate([ei0, ei1, pad]).reshape(NW * NCHUNKS, CHUNK)
  colp = jnp.concatenate([ei1, ei0, pad]).reshape(NW * NCHUNKS, CHUNK)

  hist = _hist()(rowp)
  g, v1, disb = _tc1(
      xp, hist, lin_w, lin_b.reshape(1, C),
      _eff(tc1_w1), tc1_b1.reshape(1, C),
      _eff(tc1_w2), tc1_b2.reshape(1, C),
      _eff(tc1_w3), tc1_b3.reshape(1, C))
  p1 = _hop()(rowp, colp, v1)
  tx1, v2 = _tc2(p1, disb)
  p2 = _hop()(rowp, colp, v2)
  o = _tc3(
      p2, disb, g, tx1,
      cheb_w[0], cheb_w[1], cheb_w[2], cheb_b.reshape(1, C),
      _eff(tc2_w1), tc2_b1.reshape(1, C),
      _eff(tc2_w2), tc2_b2.reshape(1, C),
      _eff(tc2_w3), tc2_b3.reshape(1, C))
  return jnp.broadcast_to(o[:N, None, :], (N, T_OUT, C))
